# baseline (speedup 1.0000x reference)
import functools

import jax
import jax.numpy as jnp
from jax import lax
from jax.experimental import pallas as pl
from jax.experimental.pallas import tpu as pltpu
from jax.experimental.pallas import tpu_sc as plsc

B = 16384
D = 32
PACK = 4
W = D * PACK
L = 16
NC = 2
NS = 16
NW = NC * NS
BPW = B // NW
CHUNK = 128
NCHUNK = BPW // CHUNK
NBLK = CHUNK // L

_mesh = plsc.VectorSubcoreMesh(core_axis_name="c", subcore_axis_name="s")


@functools.partial(
    pl.kernel,
    out_type=jax.ShapeDtypeStruct((B,), jnp.float32),
    mesh=_mesh,
    compiler_params=pltpu.CompilerParams(needs_layout_passes=False,
                                         use_tc_tiling_on_sc=False),
    scratch_types=[
        pltpu.VMEM((BPW,), jnp.int32),
        pltpu.VMEM((BPW,), jnp.int32),
        pltpu.VMEM((BPW,), jnp.int32),
        pltpu.VMEM((BPW,), jnp.int32),
        pltpu.VMEM((2, CHUNK, W), jnp.float32),
        pltpu.VMEM((2, CHUNK, W), jnp.float32),
        pltpu.VMEM((BPW,), jnp.float32),
        pltpu.VMEM((BPW,), jnp.float32),
        pltpu.VMEM((BPW,), jnp.float32),
        pltpu.SemaphoreType.DMA,
        pltpu.SemaphoreType.DMA,
        pltpu.SemaphoreType.DMA,
    ],
)
def _mf_score(uid_hbm, iid_hbm, utab_hbm, itab_hbm, ubias_hbm, ibias_hbm,
              out_hbm, idx_u, idx_i, idx4_u, idx4_i, ubuf, ibuf,
              bias_u, bias_i, out_v, sem0, sem1, semb):
    wid = lax.axis_index("s") * NC + lax.axis_index("c")
    base = wid * BPW

    pltpu.sync_copy(uid_hbm.at[pl.ds(base, BPW)], idx_u)
    pltpu.sync_copy(iid_hbm.at[pl.ds(base, BPW)], idx_i)

    for k in range(BPW // L):
        s = pl.ds(k * L, L)
        idx4_u[s] = idx_u[s] >> 2
        idx4_i[s] = idx_i[s] >> 2

    bias_copies = []
    for c in range(NCHUNK):
        s = pl.ds(c * CHUNK, CHUNK)
        bias_copies.append(pltpu.async_copy(ubias_hbm.at[idx_u.at[s]],
                                            bias_u.at[s], semb))
        bias_copies.append(pltpu.async_copy(ibias_hbm.at[idx_i.at[s]],
                                            bias_i.at[s], semb))

    sems = (sem0, sem1)

    def fire(c):
        s = pl.ds(c * CHUNK, CHUNK)
        sem = sems[c % 2]
        return (pltpu.async_copy(utab_hbm.at[idx4_u.at[s]], ubuf.at[c % 2], sem),
                pltpu.async_copy(itab_hbm.at[idx4_i.at[s]], ibuf.at[c % 2], sem))

    inflight = [fire(0), fire(1)]
    for cp in bias_copies:
        cp.wait()

    lane = lax.iota(jnp.int32, L)

    for c in range(NCHUNK):
        for cp in inflight[c]:
            cp.wait()
        ub = ubuf.at[c % 2]
        ib = ibuf.at[c % 2]
        for j in range(NBLK):
            s = pl.ds(c * CHUNK + j * L, L)
            row = lane + j * L
            col_u = (idx_u[s] & 3) << 5
            col_i = (idx_i[s] & 3) << 5
            acc = bias_u[s] + bias_i[s]
            one = jnp.ones((L,), jnp.int32)
            for d in range(D):
                acc = acc + (plsc.load_gather(ub, [row, col_u])
                             * plsc.load_gather(ib, [row, col_i]))
                if d + 1 < D:
                    col_u = col_u + one
                    col_i = col_i + one
            out_v[s] = acc
        if c + 2 < NCHUNK:
            inflight.append(fire(c + 2))

    pltpu.sync_copy(out_v, out_hbm.at[pl.ds(base, BPW)])


def kernel(user_ids, item_ids, user_table, item_table, user_bias, item_bias):
    n_u = user_table.shape[0]
    n_i = item_table.shape[0]
    ut = (user_table * jnp.float32(1.0)).reshape(n_u // PACK, W)
    it = (item_table * jnp.float32(1.0)).reshape(n_i // PACK, W)
    return _mf_score(user_ids.astype(jnp.int32), item_ids.astype(jnp.int32),
                     ut, it, user_bias.reshape(-1), item_bias.reshape(-1))
